# baseline (device time: 46870 ns/iter reference)
import jax
import jax.numpy as jnp
from jax import lax
from jax.experimental import pallas as pl
from jax.experimental.pallas import tpu as pltpu

N_DEV = 16
B, SQ, D = 2, 256, 768
DH, SKV = 64, 512
ROWS = B * SQ
CH = ROWS // N_DEV


def kernel(x, Wq, Wo, K_ext, V_ext):
    i_out = lax.axis_index("i")
    Ksl = lax.dynamic_slice_in_dim(K_ext, 2 * i_out, 2, axis=2)
    Vsl = lax.dynamic_slice_in_dim(V_ext, 2 * i_out, 2, axis=2)
    x2 = x.reshape(ROWS, D).astype(jnp.bfloat16)
    Wq = Wq.astype(jnp.bfloat16)
    Wo = Wo.astype(jnp.bfloat16)
    Ksl = Ksl.astype(jnp.bfloat16)
    Vsl = Vsl.astype(jnp.bfloat16)

    def body(x_ref, wq_ref, wo_ref, k_ref, v_ref, out_ref,
             pbuf, staging, agbuf, rs_ssem, rs_rsem, ag_ssem, ag_rsem):
        i = lax.axis_index("i")
        i3 = (i >> 3) & 1

        def store_partial(b):
            qb = jnp.dot(x_ref[pl.ds(b * SQ, SQ), :], wq_ref[...],
                         preferred_element_type=jnp.float32)
            houts = []
            for u in range(2):
                k_u = k_ref[b, :, u, :]
                v_u = v_ref[b, :, u, :]
                for t4 in range(4):
                    t = 4 * u + t4
                    qh = qb[:, t * DH:(t + 1) * DH]
                    s = lax.dot_general(
                        qh, k_u, (((1,), (1,)), ((), ())),
                        preferred_element_type=jnp.float32) * 0.125
                    m = jnp.max(s, axis=-1, keepdims=True)
                    p = jnp.exp(s - m)
                    l = jnp.sum(p, axis=-1, keepdims=True)
                    o = jnp.dot(p.astype(jnp.bfloat16), v_u,
                                preferred_element_type=jnp.float32) / l
                    houts.append(o)
            attn = jnp.concatenate(houts, axis=1)
            out_ref[pl.ds(b * SQ, SQ), :] = jnp.dot(
                attn.astype(jnp.bfloat16), wo_ref[...],
                preferred_element_type=jnp.float32)

        def rs_rdma(k):
            p = i ^ k
            return pltpu.make_async_remote_copy(
                src_ref=pbuf.at[pl.ds(p * CH, CH), :],
                dst_ref=staging.at[k],
                send_sem=rs_ssem.at[k],
                recv_sem=rs_rsem.at[k],
                device_id=(p,),
                device_id_type=pl.DeviceIdType.MESH,
            )

        def ag_rdma(k, dst_rows):
            return pltpu.make_async_remote_copy(
                src_ref=agbuf.at[pl.ds(i * CH, CH), :],
                dst_ref=agbuf.at[pl.ds(dst_rows, CH), :],
                send_sem=ag_ssem.at[k],
                recv_sem=ag_rsem.at[k],
                device_id=(i ^ k,),
                device_id_type=pl.DeviceIdType.MESH,
            )

        store_partial(0)
        pbuf[pl.ds(0, SQ), :] = out_ref[pl.ds(0, SQ), :].astype(jnp.bfloat16)
        for k in range(1, N_DEV):
            @pl.when(((k >> 3) & 1) == i3)
            def _(k=k):
                rs_rdma(k).start()

        store_partial(1)
        pbuf[pl.ds(SQ, SQ), :] = out_ref[pl.ds(SQ, SQ), :].astype(jnp.bfloat16)
        for k in range(1, N_DEV):
            @pl.when(((k >> 3) & 1) != i3)
            def _(k=k):
                rs_rdma(k).start()

        for k in range(1, N_DEV):
            rs_rdma(k).wait_recv()
        red = (out_ref[pl.ds(i * CH, CH), :]
               + staging[1:N_DEV].astype(jnp.float32).sum(axis=0))

        agbuf[pl.ds(i * CH, CH), :] = red.astype(jnp.bfloat16)
        for k in range(1, N_DEV):
            ag_rdma(k, i * CH).start()
        for k in range(1, N_DEV):
            ag_rdma(k, (i ^ k) * CH).wait_recv()
        out_ref[...] = agbuf[...].astype(jnp.float32)

        for k in range(1, N_DEV):
            rs_rdma(k).wait_send()
            ag_rdma(k, i * CH).wait_send()

    out = pl.pallas_call(
        body,
        out_shape=jax.ShapeDtypeStruct((ROWS, D), jnp.float32),
        in_specs=[pl.BlockSpec(memory_space=pltpu.VMEM)] * 5,
        out_specs=pl.BlockSpec(memory_space=pltpu.VMEM),
        scratch_shapes=[
            pltpu.VMEM((ROWS, D), jnp.bfloat16),
            pltpu.VMEM((N_DEV, CH, D), jnp.bfloat16),
            pltpu.VMEM((ROWS, D), jnp.bfloat16),
            pltpu.SemaphoreType.DMA((N_DEV,)),
            pltpu.SemaphoreType.DMA((N_DEV,)),
            pltpu.SemaphoreType.DMA((N_DEV,)),
            pltpu.SemaphoreType.DMA((N_DEV,)),
        ],
    )(x2, Wq, Wo, Ksl, Vsl)
    return out.reshape(B, SQ, D)
